# Initial kernel scaffold; baseline (speedup 1.0000x reference)
import functools

import jax
import jax.numpy as jnp
from jax import lax
from jax.experimental import pallas as pl
from jax.experimental.pallas import tpu as pltpu
from jax.experimental.pallas import tpu_sc as plsc

N = 10000
E = 320000
D = 128
NC = 2
NS = 16
NW = NC * NS
EPW = E // NW
K = 80
NCHUNK = EPW // K
RPT = N // NS

_mesh = plsc.VectorSubcoreMesh(core_axis_name="c", subcore_axis_name="s")
_f32 = jnp.float32


@functools.partial(
    pl.kernel,
    out_type=(
        jax.ShapeDtypeStruct((NC, N, D), _f32),
        jax.ShapeDtypeStruct((NC, N, 16), _f32),
    ),
    mesh=_mesh,
    scratch_types=[
        pltpu.VMEM((NCHUNK, K), jnp.int32),
        pltpu.VMEM((NCHUNK, K), jnp.int32),
        pltpu.VMEM((EPW,), _f32),
        pltpu.VMEM((K, D), _f32),
        pltpu.VMEM((K, 16), _f32),
        pltpu.VMEM_SHARED((N, D), _f32),
        pltpu.VMEM_SHARED((N, 16), _f32),
    ],
)
def _sc_pass1(x_hbm, src_hbm, dst_hbm, w_hbm, zrow_hbm, zrow16_hbm,
              ox2_hbm, odeg_hbm,
              src_v, dst_v, w_v, rows_v, ones_v, acc, dacc):
    c = lax.axis_index("c")
    s = lax.axis_index("s")
    wid = c * NS + s

    pltpu.sync_copy(src_hbm.at[wid], src_v)
    pltpu.sync_copy(dst_hbm.at[wid], dst_v)
    pltpu.sync_copy(w_hbm.at[wid], w_v)

    lane0 = jnp.where(lax.broadcasted_iota(jnp.int32, (16,), 0) == 0, 1.0, 0.0)

    def init_ones(r, carry):
        ones_v[r, :] = lane0
        return carry
    lax.fori_loop(0, K, init_ones, None)

    pltpu.sync_copy(zrow_hbm, acc.at[pl.ds(s * RPT, RPT)])
    pltpu.sync_copy(zrow16_hbm, dacc.at[pl.ds(s * RPT, RPT)])
    plsc.subcore_barrier()

    def chunk(ci, carry):
        pltpu.sync_copy(x_hbm.at[src_v.at[ci]], rows_v)

        def scale_row(r, inner):
            widx = jnp.full((16,), ci * K + r, dtype=jnp.int32)
            wv = plsc.load_gather(w_v, [widx])
            row = rows_v.at[r]
            for g in range(D // 16):
                sl = pl.ds(g * 16, 16)
                row[sl] = row[sl] * wv
            return inner
        lax.fori_loop(0, K, scale_row, None)

        pltpu.sync_copy(rows_v, acc.at[dst_v.at[ci]], add=True)
        pltpu.sync_copy(ones_v, dacc.at[dst_v.at[ci]], add=True)
        return carry
    lax.fori_loop(0, NCHUNK, chunk, None)

    plsc.subcore_barrier()
    pltpu.sync_copy(acc.at[pl.ds(s * RPT, RPT)],
                    ox2_hbm.at[c, pl.ds(s * RPT, RPT)])
    pltpu.sync_copy(dacc.at[pl.ds(s * RPT, RPT)],
                    odeg_hbm.at[c, pl.ds(s * RPT, RPT)])


@functools.partial(
    pl.kernel,
    out_type=jax.ShapeDtypeStruct((NC, N, D), _f32),
    mesh=_mesh,
    scratch_types=[
        pltpu.VMEM((NCHUNK, K), jnp.int32),
        pltpu.VMEM((NCHUNK, K), jnp.int32),
        pltpu.VMEM((K, D), _f32),
        pltpu.VMEM_SHARED((N, D), _f32),
    ],
)
def _sc_agg(a_hbm, src_hbm, dst_hbm, zrow_hbm, out_hbm,
            src_v, dst_v, rows_v, acc):
    c = lax.axis_index("c")
    s = lax.axis_index("s")
    wid = c * NS + s

    pltpu.sync_copy(src_hbm.at[wid], src_v)
    pltpu.sync_copy(dst_hbm.at[wid], dst_v)
    pltpu.sync_copy(zrow_hbm, acc.at[pl.ds(s * RPT, RPT)])
    plsc.subcore_barrier()

    def chunk(ci, carry):
        pltpu.sync_copy(a_hbm.at[src_v.at[ci]], rows_v)
        pltpu.sync_copy(rows_v, acc.at[dst_v.at[ci]], add=True)
        return carry
    lax.fori_loop(0, NCHUNK, chunk, None)

    plsc.subcore_barrier()
    pltpu.sync_copy(acc.at[pl.ds(s * RPT, RPT)],
                    out_hbm.at[c, pl.ds(s * RPT, RPT)])


_BR = 1000


def _dinv_of(pdeg_ref):
    deg = pdeg_ref[0, :, 0] + pdeg_ref[1, :, 0] + 1.0
    return lax.rsqrt(deg)


def _tc_prep_body(x_ref, px2_ref, pdeg_ref, mask_ref, o_ref):
    dinv = _dinv_of(pdeg_ref)
    x2 = px2_ref[0] + px2_ref[1]
    m = mask_ref[0, 0]
    xt = jnp.where(m[:, None] > 0, x2, x_ref[...])
    o_ref[...] = dinv[:, None] * xt


def _tc_conv_body(pm_ref, aprev_ref, pdeg_ref, w_ref, b_ref, o_ref):
    dinv = _dinv_of(pdeg_ref)
    pre = dinv[:, None] * (pm_ref[0] + pm_ref[1] + aprev_ref[...])
    z = jnp.dot(pre, w_ref[...], preferred_element_type=_f32) + b_ref[...]
    o_ref[...] = dinv[:, None] * jnp.maximum(z, 0.0)


def _tc_final_body(pm_ref, aprev_ref, pdeg_ref, w3_ref, b3_ref, batch_ref,
                   wm1_ref, bm1_ref, wm2_ref, bm2_ref, o_ref, g_acc):
    i = pl.program_id(0)
    dinv = _dinv_of(pdeg_ref)
    pre = dinv[:, None] * (pm_ref[0] + pm_ref[1] + aprev_ref[...])
    z3 = jnp.dot(pre, w3_ref[...], preferred_element_type=_f32) + b3_ref[...]
    bt = batch_ref[0, 0]
    oh = (bt[:, None] ==
          lax.broadcasted_iota(jnp.int32, (1, 64), 1)).astype(_f32)
    contrib = lax.dot_general(oh, z3, (((0,), (0,)), ((), ())),
                              preferred_element_type=_f32)

    @pl.when(i == 0)
    def _():
        g_acc[...] = contrib

    @pl.when(i > 0)
    def _():
        g_acc[...] = g_acc[...] + contrib

    @pl.when(i == pl.num_programs(0) - 1)
    def _():
        h = jnp.maximum(
            jnp.dot(g_acc[...], wm1_ref[...], preferred_element_type=_f32)
            + bm1_ref[...], 0.0)
        o_ref[...] = (jnp.dot(h, wm2_ref[...], preferred_element_type=_f32)
                      + bm2_ref[...])


def _row_spec():
    return pl.BlockSpec((_BR, D), lambda i: (i, 0))


def _part_spec(width):
    return pl.BlockSpec((NC, _BR, width), lambda i: (0, i, 0))


def _i32row_spec():
    return pl.BlockSpec((1, 1, _BR), lambda i: (i, 0, 0))


def _full(shape):
    return pl.BlockSpec(shape, lambda i: tuple(0 for _ in shape))


def _tc_prep(x, px2, pdeg, mask3):
    return pl.pallas_call(
        _tc_prep_body,
        grid=(N // _BR,),
        in_specs=[_row_spec(), _part_spec(D), _part_spec(16), _i32row_spec()],
        out_specs=_row_spec(),
        out_shape=jax.ShapeDtypeStruct((N, D), _f32),
    )(x, px2, pdeg, mask3)


def _tc_conv(pm, aprev, pdeg, w, b2d):
    return pl.pallas_call(
        _tc_conv_body,
        grid=(N // _BR,),
        in_specs=[_part_spec(D), _row_spec(), _part_spec(16),
                  _full((D, D)), _full((1, D))],
        out_specs=_row_spec(),
        out_shape=jax.ShapeDtypeStruct((N, D), _f32),
    )(pm, aprev, pdeg, w, b2d)


def _tc_final(pm, aprev, pdeg, w3, b3, batch3, wm1, bm1, wm2, bm2):
    return pl.pallas_call(
        _tc_final_body,
        grid=(N // _BR,),
        in_specs=[_part_spec(D), _row_spec(), _part_spec(16),
                  _full((D, D)), _full((1, D)), _i32row_spec(),
                  _full((D, D)), _full((1, D)), _full((D, 10)),
                  _full((1, 10))],
        out_specs=_full((64, 10)),
        out_shape=jax.ShapeDtypeStruct((64, 10), _f32),
        scratch_shapes=[pltpu.VMEM((64, D), _f32)],
    )(pm, aprev, pdeg, w3, b3, batch3, wm1, bm1, wm2, bm2)


def kernel(x, edge_index, supernode_mask, edge_mask, batch,
           W1, b1, W2, b2, W3, b3, Wm1, bm1, Wm2, bm2):
    src3 = edge_index[0].reshape(NW, NCHUNK, K)
    dst3 = edge_index[1].reshape(NW, NCHUNK, K)
    w2 = edge_mask.reshape(NW, EPW)
    mask3 = supernode_mask.astype(jnp.int32).reshape(N // _BR, 1, _BR)
    batch3 = batch.reshape(N // _BR, 1, _BR)
    zrow = jnp.zeros((RPT, D), _f32)
    zrow16 = jnp.zeros((RPT, 16), _f32)
    b1r, b2r, b3r = b1.reshape(1, D), b2.reshape(1, D), b3.reshape(1, D)
    bm1r, bm2r = bm1.reshape(1, D), bm2.reshape(1, 10)

    px2, pdeg = _sc_pass1(x, src3, dst3, w2, zrow, zrow16)
    a1 = _tc_prep(x, px2, pdeg, mask3)
    m1 = _sc_agg(a1, src3, dst3, zrow)
    a2 = _tc_conv(m1, a1, pdeg, W1, b1r)
    m2 = _sc_agg(a2, src3, dst3, zrow)
    a3 = _tc_conv(m2, a2, pdeg, W2, b2r)
    m3 = _sc_agg(a3, src3, dst3, zrow)
    return _tc_final(m3, a3, pdeg, W3, b3r, batch3, Wm1, bm1r, Wm2, bm2r)


# speedup vs baseline: 5.1882x; 5.1882x over previous
import functools

import jax
import jax.numpy as jnp
from jax import lax
from jax.experimental import pallas as pl
from jax.experimental.pallas import tpu as pltpu
from jax.experimental.pallas import tpu_sc as plsc

N = 10000
E = 320000
D = 128
NC = 2
NS = 16
NW = NC * NS
K = 128
NCHUNK = 80
SUP = 8
NSUP = NCHUNK // SUP
EPW = NCHUNK * K
EPAD = NW * EPW
NPAD = 10240
RPT = NPAD // NS

_mesh = plsc.VectorSubcoreMesh(core_axis_name="c", subcore_axis_name="s")
_f32 = jnp.float32


@functools.partial(
    pl.kernel,
    out_type=jax.ShapeDtypeStruct((NC, NPAD, D), _f32),
    mesh=_mesh,
    scratch_types=[
        pltpu.VMEM((SUP, K), jnp.int32),
        pltpu.VMEM((SUP, K), jnp.int32),
        pltpu.VMEM((SUP, K), _f32),
        pltpu.VMEM((K, D), _f32),
        pltpu.VMEM_SHARED((NPAD, D), _f32),
    ],
)
def _sc_pass1(x_hbm, src_hbm, dst_hbm, w_hbm, zrow_hbm, ox2_hbm,
              src_v, dst_v, w_v, rows_v, acc):
    c = lax.axis_index("c")
    s = lax.axis_index("s")
    wid = c * NS + s

    pltpu.sync_copy(zrow_hbm, acc.at[pl.ds(s * RPT, RPT)])
    plsc.subcore_barrier()

    def sup(cs, carry):
        pltpu.sync_copy(src_hbm.at[wid, pl.ds(cs * SUP, SUP)], src_v)
        pltpu.sync_copy(dst_hbm.at[wid, pl.ds(cs * SUP, SUP)], dst_v)
        pltpu.sync_copy(w_hbm.at[wid, pl.ds(cs * SUP, SUP)], w_v)

        def chunk(cj, inner):
            pltpu.sync_copy(x_hbm.at[src_v.at[cj]], rows_v)
            w_row = w_v.at[cj]

            def scale_group(g16, inner2):
                wvec = w_row[pl.ds(g16 * 16, 16)]
                for l in range(16):
                    wl = wvec[l]
                    row = rows_v.at[g16 * 16 + l]
                    for g in range(D // 16):
                        sl = pl.ds(g * 16, 16)
                        row[sl] = row[sl] * wl
                return inner2
            lax.fori_loop(0, K // 16, scale_group, None)

            pltpu.sync_copy(rows_v, acc.at[dst_v.at[cj]], add=True)
            return inner
        lax.fori_loop(0, SUP, chunk, None)
        return carry
    lax.fori_loop(0, NSUP, sup, None)

    plsc.subcore_barrier()
    pltpu.sync_copy(acc.at[pl.ds(s * RPT, RPT)],
                    ox2_hbm.at[c, pl.ds(s * RPT, RPT)])


@functools.partial(
    pl.kernel,
    out_type=jax.ShapeDtypeStruct((NC, NPAD, D), _f32),
    mesh=_mesh,
    scratch_types=[
        pltpu.VMEM((SUP, K), jnp.int32),
        pltpu.VMEM((K, D), _f32),
        pltpu.VMEM_SHARED((NPAD, D), _f32),
    ],
)
def _sc_deg(dst_hbm, zrow_hbm, odeg_hbm, dst_v, ones_v, dacc):
    c = lax.axis_index("c")
    s = lax.axis_index("s")
    wid = c * NS + s

    lane0 = jnp.where(lax.broadcasted_iota(jnp.int32, (16,), 0) == 0, 1.0, 0.0)
    zero16 = jnp.zeros((16,), _f32)

    def init_ones(r, carry):
        ones_v[r, pl.ds(0, 16)] = lane0
        for g in range(1, D // 16):
            ones_v[r, pl.ds(g * 16, 16)] = zero16
        return carry
    lax.fori_loop(0, K, init_ones, None)

    pltpu.sync_copy(zrow_hbm, dacc.at[pl.ds(s * RPT, RPT)])
    plsc.subcore_barrier()

    def sup(cs, carry):
        pltpu.sync_copy(dst_hbm.at[wid, pl.ds(cs * SUP, SUP)], dst_v)

        def chunk(cj, inner):
            pltpu.sync_copy(ones_v, dacc.at[dst_v.at[cj]], add=True)
            return inner
        lax.fori_loop(0, SUP, chunk, None)
        return carry
    lax.fori_loop(0, NSUP, sup, None)

    plsc.subcore_barrier()
    pltpu.sync_copy(dacc.at[pl.ds(s * RPT, RPT)],
                    odeg_hbm.at[c, pl.ds(s * RPT, RPT)])


@functools.partial(
    pl.kernel,
    out_type=jax.ShapeDtypeStruct((NC, NPAD, D), _f32),
    mesh=_mesh,
    scratch_types=[
        pltpu.VMEM((SUP, K), jnp.int32),
        pltpu.VMEM((SUP, K), jnp.int32),
        pltpu.VMEM((K, D), _f32),
        pltpu.VMEM_SHARED((NPAD, D), _f32),
    ],
)
def _sc_agg(a_hbm, src_hbm, dst_hbm, zrow_hbm, out_hbm,
            src_v, dst_v, rows_v, acc):
    c = lax.axis_index("c")
    s = lax.axis_index("s")
    wid = c * NS + s

    pltpu.sync_copy(zrow_hbm, acc.at[pl.ds(s * RPT, RPT)])
    plsc.subcore_barrier()

    def sup(cs, carry):
        pltpu.sync_copy(src_hbm.at[wid, pl.ds(cs * SUP, SUP)], src_v)
        pltpu.sync_copy(dst_hbm.at[wid, pl.ds(cs * SUP, SUP)], dst_v)

        def chunk(cj, inner):
            pltpu.sync_copy(a_hbm.at[src_v.at[cj]], rows_v)
            pltpu.sync_copy(rows_v, acc.at[dst_v.at[cj]], add=True)
            return inner
        lax.fori_loop(0, SUP, chunk, None)
        return carry
    lax.fori_loop(0, NSUP, sup, None)

    plsc.subcore_barrier()
    pltpu.sync_copy(acc.at[pl.ds(s * RPT, RPT)],
                    out_hbm.at[c, pl.ds(s * RPT, RPT)])


_BR = 1000


def _dinv_of(pdeg_ref):
    deg = pdeg_ref[0, :, 0] + pdeg_ref[1, :, 0] + 1.0
    return lax.rsqrt(deg)


def _tc_prep_body(x_ref, px2_ref, pdeg_ref, mask_ref, o_ref):
    dinv = _dinv_of(pdeg_ref)
    x2 = px2_ref[0] + px2_ref[1]
    m = mask_ref[0, 0]
    xt = jnp.where(m[:, None] > 0, x2, x_ref[...])
    o_ref[...] = dinv[:, None] * xt


def _tc_conv_body(pm_ref, aprev_ref, pdeg_ref, w_ref, b_ref, o_ref):
    dinv = _dinv_of(pdeg_ref)
    pre = dinv[:, None] * (pm_ref[0] + pm_ref[1] + aprev_ref[...])
    z = jnp.dot(pre, w_ref[...], preferred_element_type=_f32) + b_ref[...]
    o_ref[...] = dinv[:, None] * jnp.maximum(z, 0.0)


def _tc_final_body(pm_ref, aprev_ref, pdeg_ref, w3_ref, b3_ref, batch_ref,
                   wm1_ref, bm1_ref, wm2_ref, bm2_ref, o_ref, g_acc):
    i = pl.program_id(0)
    dinv = _dinv_of(pdeg_ref)
    pre = dinv[:, None] * (pm_ref[0] + pm_ref[1] + aprev_ref[...])
    z3 = jnp.dot(pre, w3_ref[...], preferred_element_type=_f32) + b3_ref[...]
    bt = batch_ref[0, 0]
    oh = (bt[:, None] ==
          lax.broadcasted_iota(jnp.int32, (1, 64), 1)).astype(_f32)
    contrib = lax.dot_general(oh, z3, (((0,), (0,)), ((), ())),
                              preferred_element_type=_f32)

    @pl.when(i == 0)
    def _():
        g_acc[...] = contrib

    @pl.when(i > 0)
    def _():
        g_acc[...] = g_acc[...] + contrib

    @pl.when(i == pl.num_programs(0) - 1)
    def _():
        h = jnp.maximum(
            jnp.dot(g_acc[...], wm1_ref[...], preferred_element_type=_f32)
            + bm1_ref[...], 0.0)
        o_ref[...] = (jnp.dot(h, wm2_ref[...], preferred_element_type=_f32)
                      + bm2_ref[...])


def _row_spec():
    return pl.BlockSpec((_BR, D), lambda i: (i, 0))


def _part_spec(width):
    return pl.BlockSpec((NC, _BR, width), lambda i: (0, i, 0))


def _i32row_spec():
    return pl.BlockSpec((1, 1, _BR), lambda i: (i, 0, 0))


def _full(shape):
    return pl.BlockSpec(shape, lambda i: tuple(0 for _ in shape))


def _tc_prep(x, px2, pdeg, mask3):
    return pl.pallas_call(
        _tc_prep_body,
        grid=(N // _BR,),
        in_specs=[_row_spec(), _part_spec(D), _part_spec(D), _i32row_spec()],
        out_specs=_row_spec(),
        out_shape=jax.ShapeDtypeStruct((N, D), _f32),
    )(x, px2, pdeg, mask3)


def _tc_conv(pm, aprev, pdeg, w, b2d):
    return pl.pallas_call(
        _tc_conv_body,
        grid=(N // _BR,),
        in_specs=[_part_spec(D), _row_spec(), _part_spec(D),
                  _full((D, D)), _full((1, D))],
        out_specs=_row_spec(),
        out_shape=jax.ShapeDtypeStruct((N, D), _f32),
    )(pm, aprev, pdeg, w, b2d)


def _tc_final(pm, aprev, pdeg, w3, b3, batch3, wm1, bm1, wm2, bm2):
    return pl.pallas_call(
        _tc_final_body,
        grid=(N // _BR,),
        in_specs=[_part_spec(D), _row_spec(), _part_spec(D),
                  _full((D, D)), _full((1, D)), _i32row_spec(),
                  _full((D, D)), _full((1, D)), _full((D, 10)),
                  _full((1, 10))],
        out_specs=_full((64, 10)),
        out_shape=jax.ShapeDtypeStruct((64, 10), _f32),
        scratch_shapes=[pltpu.VMEM((64, D), _f32)],
    )(pm, aprev, pdeg, w3, b3, batch3, wm1, bm1, wm2, bm2)


def kernel(x, edge_index, supernode_mask, edge_mask, batch,
           W1, b1, W2, b2, W3, b3, Wm1, bm1, Wm2, bm2):
    pad = EPAD - E
    src3 = jnp.concatenate(
        [edge_index[0], jnp.zeros((pad,), jnp.int32)]).reshape(NW, NCHUNK, K)
    dst3 = jnp.concatenate(
        [edge_index[1], jnp.full((pad,), N, jnp.int32)]).reshape(NW, NCHUNK, K)
    w3 = jnp.concatenate(
        [edge_mask, jnp.zeros((pad,), _f32)]).reshape(NW, NCHUNK, K)
    mask3 = supernode_mask.astype(jnp.int32).reshape(N // _BR, 1, _BR)
    batch3 = batch.reshape(N // _BR, 1, _BR)
    zrow = jnp.zeros((RPT, D), _f32)
    b1r, b2r, b3r = b1.reshape(1, D), b2.reshape(1, D), b3.reshape(1, D)
    bm1r, bm2r = bm1.reshape(1, D), bm2.reshape(1, 10)

    px2 = _sc_pass1(x, src3, dst3, w3, zrow)
    pdeg = _sc_deg(dst3, zrow)
    a1 = _tc_prep(x, px2, pdeg, mask3)
    m1 = _sc_agg(a1, src3, dst3, zrow)
    a2 = _tc_conv(m1, a1, pdeg, W1, b1r)
    m2 = _sc_agg(a2, src3, dst3, zrow)
    a3 = _tc_conv(m2, a2, pdeg, W2, b2r)
    m3 = _sc_agg(a3, src3, dst3, zrow)
    return _tc_final(m3, a3, pdeg, W3, b3r, batch3, Wm1, bm1r, Wm2, bm2r)


# speedup vs baseline: 5.9416x; 1.1452x over previous
import functools

import jax
import jax.numpy as jnp
from jax import lax
from jax.experimental import pallas as pl
from jax.experimental.pallas import tpu as pltpu
from jax.experimental.pallas import tpu_sc as plsc

N = 10000
E = 320000
D = 128
NC = 2
NS = 16
NW = NC * NS
K = 128
NCHUNK = 80
SUP = 8
NSUP = NCHUNK // SUP
EPW = NCHUNK * K
EPAD = NW * EPW
NPAD = 10240
RPT = NPAD // NS

_mesh = plsc.VectorSubcoreMesh(core_axis_name="c", subcore_axis_name="s")
_f32 = jnp.float32


def _make_agg(weighted):
    scratch = [
        pltpu.VMEM((SUP, K), jnp.int32),
        pltpu.VMEM((SUP, K), jnp.int32),
        pltpu.VMEM((SUP, K), jnp.int32),
        pltpu.VMEM((SUP, K), jnp.int32),
        pltpu.VMEM((K, D), _f32),
        pltpu.VMEM((K, D), _f32),
        pltpu.SemaphoreType.DMA,
        pltpu.SemaphoreType.DMA,
        pltpu.SemaphoreType.DMA,
        pltpu.SemaphoreType.DMA,
        pltpu.VMEM_SHARED((NPAD, D), _f32),
    ]
    if weighted:
        scratch = [pltpu.VMEM((SUP, K), _f32),
                   pltpu.VMEM((SUP, K), _f32)] + scratch

    def body(x_hbm, src_hbm, dst_hbm, w_hbm, zrow_hbm, out_hbm,
             wA, wB, srcA, dstA, srcB, dstB, rows0, rows1,
             gs0, gs1, ssA, ssB, acc):
        c = lax.axis_index("c")
        s = lax.axis_index("s")
        wid = c * NS + s
        rows = (rows0, rows1)
        gsem = (gs0, gs1)

        def stage(sb, dst_idx, dst_src, dst_dst, dst_w, sem):
            pltpu.async_copy(src_hbm.at[wid, pl.ds(sb * SUP, SUP)],
                             dst_src, sem)
            pltpu.async_copy(dst_hbm.at[wid, pl.ds(sb * SUP, SUP)],
                             dst_dst, sem)
            if weighted:
                pltpu.async_copy(w_hbm.at[wid, pl.ds(sb * SUP, SUP)],
                                 dst_w, sem)

        def stage_wait(dst_src, dst_dst, dst_w, sem):
            pltpu.make_async_copy(src_hbm.at[0, pl.ds(0, SUP)],
                                  dst_src, sem).wait()
            pltpu.make_async_copy(src_hbm.at[0, pl.ds(0, SUP)],
                                  dst_dst, sem).wait()
            if weighted:
                pltpu.make_async_copy(w_hbm.at[0, pl.ds(0, SUP)],
                                      dst_w, sem).wait()

        def fire_gather(idx_row, buf, sem):
            pltpu.async_copy(x_hbm.at[idx_row], buf, sem)

        def wait_gather(buf, sem):
            pltpu.make_async_copy(x_hbm.at[pl.ds(0, K)], buf, sem).wait()

        def scale(buf, w_sb, cj):
            w_row = w_sb.at[cj]

            def scale_group(g16, inner2):
                wvec = w_row[pl.ds(g16 * 16, 16)]
                for l in range(16):
                    wl = wvec[l]
                    row = buf.at[g16 * 16 + l]
                    for g in range(D // 16):
                        sl = pl.ds(g * 16, 16)
                        row[sl] = row[sl] * wl
                return inner2
            lax.fori_loop(0, K // 16, scale_group, None)

        pltpu.sync_copy(zrow_hbm, acc.at[pl.ds(s * RPT, RPT)])
        pltpu.sync_copy(src_hbm.at[wid, pl.ds(0, SUP)], srcA)
        pltpu.sync_copy(dst_hbm.at[wid, pl.ds(0, SUP)], dstA)
        if weighted:
            pltpu.sync_copy(w_hbm.at[wid, pl.ds(0, SUP)], wA)
        plsc.subcore_barrier()
        fire_gather(srcA.at[0], rows0, gs0)

        def half(cp, sb_dyn, src_sb, dst_sb, w_sb, o_src, o_dst, o_w,
                 o_sem, stage_next, next_guard):
            for j in range(SUP):
                cur = rows[j % 2]
                if j < SUP - 1:
                    fire_gather(src_sb.at[j + 1], rows[(j + 1) % 2],
                                gsem[(j + 1) % 2])
                else:
                    @pl.when(next_guard)
                    def _():
                        stage_wait(o_src, o_dst, o_w, o_sem)
                        fire_gather(o_src.at[0], rows[0], gsem[0])
                wait_gather(cur, gsem[j % 2])
                if weighted:
                    scale(cur, w_sb, j)
                pltpu.sync_copy(cur, acc.at[dst_sb.at[j]], add=True)

        def pair(cp, carry):
            true_ = cp >= 0
            stage(2 * cp + 1, None, srcB, dstB, wB, ssB)
            half(cp, 2 * cp, srcA, dstA, wA, srcB, dstB, wB, ssB,
                 None, true_)
            @pl.when(cp < NSUP // 2 - 1)
            def _():
                stage(2 * cp + 2, None, srcA, dstA, wA, ssA)
            half(cp, 2 * cp + 1, srcB, dstB, wB, srcA, dstA, wA, ssA,
                 None, cp < NSUP // 2 - 1)
            return carry
        lax.fori_loop(0, NSUP // 2, pair, None)

        plsc.subcore_barrier()
        pltpu.sync_copy(acc.at[pl.ds(s * RPT, RPT)],
                        out_hbm.at[c, pl.ds(s * RPT, RPT)])

    if weighted:
        def wbody(x_hbm, src_hbm, dst_hbm, w_hbm, zrow_hbm, out_hbm,
                  wA, wB, srcA, dstA, srcB, dstB, rows0, rows1,
                  gs0, gs1, ssA, ssB, acc):
            body(x_hbm, src_hbm, dst_hbm, w_hbm, zrow_hbm, out_hbm,
                 wA, wB, srcA, dstA, srcB, dstB, rows0, rows1,
                 gs0, gs1, ssA, ssB, acc)
        fn = wbody
    else:
        def ubody(x_hbm, src_hbm, dst_hbm, zrow_hbm, out_hbm,
                  srcA, dstA, srcB, dstB, rows0, rows1,
                  gs0, gs1, ssA, ssB, acc):
            body(x_hbm, src_hbm, dst_hbm, None, zrow_hbm, out_hbm,
                 None, None, srcA, dstA, srcB, dstB, rows0, rows1,
                 gs0, gs1, ssA, ssB, acc)
        fn = ubody
    return pl.kernel(
        fn,
        out_type=jax.ShapeDtypeStruct((NC, NPAD, D), _f32),
        mesh=_mesh,
        scratch_types=scratch,
    )


_sc_pass1 = _make_agg(weighted=True)
_sc_agg_p = _make_agg(weighted=False)


@functools.partial(
    pl.kernel,
    out_type=jax.ShapeDtypeStruct((NC, NPAD, D), _f32),
    mesh=_mesh,
    scratch_types=[
        pltpu.VMEM((SUP, K), jnp.int32),
        pltpu.VMEM((K, D), _f32),
        pltpu.VMEM_SHARED((NPAD, D), _f32),
    ],
)
def _sc_deg(dst_hbm, zrow_hbm, odeg_hbm, dst_v, ones_v, dacc):
    c = lax.axis_index("c")
    s = lax.axis_index("s")
    wid = c * NS + s

    lane0 = jnp.where(lax.broadcasted_iota(jnp.int32, (16,), 0) == 0, 1.0, 0.0)
    zero16 = jnp.zeros((16,), _f32)

    def init_ones(r, carry):
        ones_v[r, pl.ds(0, 16)] = lane0
        for g in range(1, D // 16):
            ones_v[r, pl.ds(g * 16, 16)] = zero16
        return carry
    lax.fori_loop(0, K, init_ones, None)

    pltpu.sync_copy(zrow_hbm, dacc.at[pl.ds(s * RPT, RPT)])
    plsc.subcore_barrier()

    def sup(cs, carry):
        pltpu.sync_copy(dst_hbm.at[wid, pl.ds(cs * SUP, SUP)], dst_v)

        def chunk(cj, inner):
            pltpu.sync_copy(ones_v, dacc.at[dst_v.at[cj]], add=True)
            return inner
        lax.fori_loop(0, SUP, chunk, None)
        return carry
    lax.fori_loop(0, NSUP, sup, None)

    plsc.subcore_barrier()
    pltpu.sync_copy(dacc.at[pl.ds(s * RPT, RPT)],
                    odeg_hbm.at[c, pl.ds(s * RPT, RPT)])


_BR = 1000


def _dinv_of(pdeg_ref):
    deg = pdeg_ref[0, :, 0] + pdeg_ref[1, :, 0] + 1.0
    return lax.rsqrt(deg)


def _tc_prep_body(x_ref, px2_ref, pdeg_ref, mask_ref, o_ref):
    dinv = _dinv_of(pdeg_ref)
    x2 = px2_ref[0] + px2_ref[1]
    m = mask_ref[0, 0]
    xt = jnp.where(m[:, None] > 0, x2, x_ref[...])
    o_ref[...] = dinv[:, None] * xt


def _tc_conv_body(pm_ref, aprev_ref, pdeg_ref, w_ref, b_ref, o_ref):
    dinv = _dinv_of(pdeg_ref)
    pre = dinv[:, None] * (pm_ref[0] + pm_ref[1] + aprev_ref[...])
    z = jnp.dot(pre, w_ref[...], preferred_element_type=_f32) + b_ref[...]
    o_ref[...] = dinv[:, None] * jnp.maximum(z, 0.0)


def _tc_final_body(pm_ref, aprev_ref, pdeg_ref, w3_ref, b3_ref, batch_ref,
                   wm1_ref, bm1_ref, wm2_ref, bm2_ref, o_ref, g_acc):
    i = pl.program_id(0)
    dinv = _dinv_of(pdeg_ref)
    pre = dinv[:, None] * (pm_ref[0] + pm_ref[1] + aprev_ref[...])
    z3 = jnp.dot(pre, w3_ref[...], preferred_element_type=_f32) + b3_ref[...]
    bt = batch_ref[0, 0]
    oh = (bt[:, None] ==
          lax.broadcasted_iota(jnp.int32, (1, 64), 1)).astype(_f32)
    contrib = lax.dot_general(oh, z3, (((0,), (0,)), ((), ())),
                              preferred_element_type=_f32)

    @pl.when(i == 0)
    def _():
        g_acc[...] = contrib

    @pl.when(i > 0)
    def _():
        g_acc[...] = g_acc[...] + contrib

    @pl.when(i == pl.num_programs(0) - 1)
    def _():
        h = jnp.maximum(
            jnp.dot(g_acc[...], wm1_ref[...], preferred_element_type=_f32)
            + bm1_ref[...], 0.0)
        o_ref[...] = (jnp.dot(h, wm2_ref[...], preferred_element_type=_f32)
                      + bm2_ref[...])


def _row_spec():
    return pl.BlockSpec((_BR, D), lambda i: (i, 0))


def _part_spec(width):
    return pl.BlockSpec((NC, _BR, width), lambda i: (0, i, 0))


def _i32row_spec():
    return pl.BlockSpec((1, 1, _BR), lambda i: (i, 0, 0))


def _full(shape):
    return pl.BlockSpec(shape, lambda i: tuple(0 for _ in shape))


def _tc_prep(x, px2, pdeg, mask3):
    return pl.pallas_call(
        _tc_prep_body,
        grid=(N // _BR,),
        in_specs=[_row_spec(), _part_spec(D), _part_spec(D), _i32row_spec()],
        out_specs=_row_spec(),
        out_shape=jax.ShapeDtypeStruct((N, D), _f32),
    )(x, px2, pdeg, mask3)


def _tc_conv(pm, aprev, pdeg, w, b2d):
    return pl.pallas_call(
        _tc_conv_body,
        grid=(N // _BR,),
        in_specs=[_part_spec(D), _row_spec(), _part_spec(D),
                  _full((D, D)), _full((1, D))],
        out_specs=_row_spec(),
        out_shape=jax.ShapeDtypeStruct((N, D), _f32),
    )(pm, aprev, pdeg, w, b2d)


def _tc_final(pm, aprev, pdeg, w3, b3, batch3, wm1, bm1, wm2, bm2):
    return pl.pallas_call(
        _tc_final_body,
        grid=(N // _BR,),
        in_specs=[_part_spec(D), _row_spec(), _part_spec(D),
                  _full((D, D)), _full((1, D)), _i32row_spec(),
                  _full((D, D)), _full((1, D)), _full((D, 10)),
                  _full((1, 10))],
        out_specs=_full((64, 10)),
        out_shape=jax.ShapeDtypeStruct((64, 10), _f32),
        scratch_shapes=[pltpu.VMEM((64, D), _f32)],
    )(pm, aprev, pdeg, w3, b3, batch3, wm1, bm1, wm2, bm2)


def kernel(x, edge_index, supernode_mask, edge_mask, batch,
           W1, b1, W2, b2, W3, b3, Wm1, bm1, Wm2, bm2):
    pad = EPAD - E
    src3 = jnp.concatenate(
        [edge_index[0], jnp.zeros((pad,), jnp.int32)]).reshape(NW, NCHUNK, K)
    dst3 = jnp.concatenate(
        [edge_index[1], jnp.full((pad,), N, jnp.int32)]).reshape(NW, NCHUNK, K)
    w3 = jnp.concatenate(
        [edge_mask, jnp.zeros((pad,), _f32)]).reshape(NW, NCHUNK, K)
    mask3 = supernode_mask.astype(jnp.int32).reshape(N // _BR, 1, _BR)
    batch3 = batch.reshape(N // _BR, 1, _BR)
    zrow = jnp.zeros((RPT, D), _f32)
    b1r, b2r, b3r = b1.reshape(1, D), b2.reshape(1, D), b3.reshape(1, D)
    bm1r, bm2r = bm1.reshape(1, D), bm2.reshape(1, 10)

    px2 = _sc_pass1(x, src3, dst3, w3, zrow)
    pdeg = _sc_deg(dst3, zrow)
    a1 = _tc_prep(x, px2, pdeg, mask3)
    m1 = _sc_agg_p(a1, src3, dst3, zrow)
    a2 = _tc_conv(m1, a1, pdeg, W1, b1r)
    m2 = _sc_agg_p(a2, src3, dst3, zrow)
    a3 = _tc_conv(m2, a2, pdeg, W2, b2r)
    m3 = _sc_agg_p(a3, src3, dst3, zrow)
    return _tc_final(m3, a3, pdeg, W3, b3r, batch3, Wm1, bm1r, Wm2, bm2r)


# speedup vs baseline: 6.1846x; 1.0409x over previous
import functools

import jax
import jax.numpy as jnp
from jax import lax
from jax.experimental import pallas as pl
from jax.experimental.pallas import tpu as pltpu
from jax.experimental.pallas import tpu_sc as plsc

N = 10000
E = 320000
D = 128
NC = 2
NS = 16
NW = NC * NS
K = 64
NCHUNK = 160
NBUF = 4
SUP = 8
NSUP = NCHUNK // SUP
EPW = NCHUNK * K
EPAD = NW * EPW
NPAD = 10240
RPT = NPAD // NS

_mesh = plsc.VectorSubcoreMesh(core_axis_name="c", subcore_axis_name="s")
_f32 = jnp.float32


def _make_agg(weighted):
    scratch = [
        pltpu.VMEM((SUP, K), jnp.int32),
        pltpu.VMEM((SUP, K), jnp.int32),
        pltpu.VMEM((SUP, K), jnp.int32),
        pltpu.VMEM((SUP, K), jnp.int32),
    ] + [pltpu.VMEM((K, D), _f32) for _ in range(NBUF)] \
      + [pltpu.SemaphoreType.DMA for _ in range(NBUF)] + [
        pltpu.SemaphoreType.DMA,
        pltpu.SemaphoreType.DMA,
        pltpu.VMEM_SHARED((NPAD, D), _f32),
    ]
    if weighted:
        scratch = [pltpu.VMEM((SUP, K), _f32),
                   pltpu.VMEM((SUP, K), _f32)] + scratch

    def body(x_hbm, src_hbm, dst_hbm, w_hbm, zrow_hbm, out_hbm,
             wA, wB, srcA, dstA, srcB, dstB, rows, gsem, ssA, ssB, acc):
        c = lax.axis_index("c")
        s = lax.axis_index("s")
        wid = c * NS + s
        AB = ((srcA, dstA, wA, ssA), (srcB, dstB, wB, ssB))

        def stage(sb, bufs):
            sv, dv, wv, sem = bufs
            pltpu.async_copy(src_hbm.at[wid, pl.ds(sb * SUP, SUP)], sv, sem)
            pltpu.async_copy(dst_hbm.at[wid, pl.ds(sb * SUP, SUP)], dv, sem)
            if weighted:
                pltpu.async_copy(w_hbm.at[wid, pl.ds(sb * SUP, SUP)], wv, sem)

        def stage_wait(bufs):
            sv, dv, wv, sem = bufs
            pltpu.make_async_copy(src_hbm.at[0, pl.ds(0, SUP)], sv, sem).wait()
            pltpu.make_async_copy(src_hbm.at[0, pl.ds(0, SUP)], dv, sem).wait()
            if weighted:
                pltpu.make_async_copy(w_hbm.at[0, pl.ds(0, SUP)],
                                      wv, sem).wait()

        def fire(idx_row, b):
            pltpu.async_copy(x_hbm.at[idx_row], rows[b], gsem[b])

        def wait_gather(b):
            pltpu.make_async_copy(x_hbm.at[pl.ds(0, K)], rows[b],
                                  gsem[b]).wait()

        def scale(buf, w_sb, cj):
            w_row = w_sb.at[cj]

            def scale_group(g16, inner2):
                wvec = w_row[pl.ds(g16 * 16, 16)]
                for l in range(16):
                    wl = wvec[l]
                    row = buf.at[g16 * 16 + l]
                    for g in range(D // 16):
                        sl = pl.ds(g * 16, 16)
                        row[sl] = row[sl] * wl
                return inner2
            lax.fori_loop(0, K // 16, scale_group, None)

        pltpu.sync_copy(zrow_hbm, acc.at[pl.ds(s * RPT, RPT)])
        pltpu.sync_copy(src_hbm.at[wid, pl.ds(0, SUP)], srcA)
        pltpu.sync_copy(dst_hbm.at[wid, pl.ds(0, SUP)], dstA)
        if weighted:
            pltpu.sync_copy(w_hbm.at[wid, pl.ds(0, SUP)], wA)
        plsc.subcore_barrier()
        for t in range(NBUF - 1):
            fire(srcA.at[t], t)

        def half(base, cur_ix, nxt_ix, guard):
            sv, dv, wv, _ = cur_ix
            for j in range(SUP):
                bc = j % NBUF
                bn = (j + NBUF - 1) % NBUF
                ahead = j + NBUF - 1
                if ahead < SUP:
                    fire(sv.at[ahead], bn)
                else:
                    @pl.when(guard)
                    def _():
                        if ahead == SUP:
                            stage_wait(nxt_ix)
                        fire(nxt_ix[0].at[ahead - SUP], bn)
                wait_gather(bc)
                if weighted:
                    scale(rows[bc], wv, j)
                pltpu.sync_copy(rows[bc], acc.at[dv.at[j]], add=True)

        def pair(cp, carry):
            always = cp >= 0
            more = cp < NSUP // 2 - 1
            stage(2 * cp + 1, AB[1])
            half(16 * cp, AB[0], AB[1], always)

            @pl.when(more)
            def _():
                stage(2 * cp + 2, AB[0])
            half(16 * cp + 8, AB[1], AB[0], more)
            return carry
        lax.fori_loop(0, NSUP // 2, pair, None)

        plsc.subcore_barrier()
        pltpu.sync_copy(acc.at[pl.ds(s * RPT, RPT)],
                        out_hbm.at[c, pl.ds(s * RPT, RPT)])

    if weighted:
        def wbody(x_hbm, src_hbm, dst_hbm, w_hbm, zrow_hbm, out_hbm,
                  wA, wB, srcA, dstA, srcB, dstB,
                  r0, r1, r2, r3, g0, g1, g2, g3, ssA, ssB, acc):
            body(x_hbm, src_hbm, dst_hbm, w_hbm, zrow_hbm, out_hbm,
                 wA, wB, srcA, dstA, srcB, dstB,
                 (r0, r1, r2, r3), (g0, g1, g2, g3), ssA, ssB, acc)
        fn = wbody
    else:
        def ubody(x_hbm, src_hbm, dst_hbm, zrow_hbm, out_hbm,
                  srcA, dstA, srcB, dstB,
                  r0, r1, r2, r3, g0, g1, g2, g3, ssA, ssB, acc):
            body(x_hbm, src_hbm, dst_hbm, None, zrow_hbm, out_hbm,
                 None, None, srcA, dstA, srcB, dstB,
                 (r0, r1, r2, r3), (g0, g1, g2, g3), ssA, ssB, acc)
        fn = ubody
    return pl.kernel(
        fn,
        out_type=jax.ShapeDtypeStruct((NC, NPAD, D), _f32),
        mesh=_mesh,
        scratch_types=scratch,
    )


_sc_pass1 = _make_agg(weighted=True)
_sc_agg_p = _make_agg(weighted=False)


@functools.partial(
    pl.kernel,
    out_type=jax.ShapeDtypeStruct((NC, NPAD, D), _f32),
    mesh=_mesh,
    scratch_types=[
        pltpu.VMEM((SUP, K), jnp.int32),
        pltpu.VMEM((K, D), _f32),
        pltpu.VMEM_SHARED((NPAD, D), _f32),
    ],
)
def _sc_deg(dst_hbm, zrow_hbm, odeg_hbm, dst_v, ones_v, dacc):
    c = lax.axis_index("c")
    s = lax.axis_index("s")
    wid = c * NS + s

    lane0 = jnp.where(lax.broadcasted_iota(jnp.int32, (16,), 0) == 0, 1.0, 0.0)
    zero16 = jnp.zeros((16,), _f32)

    def init_ones(r, carry):
        ones_v[r, pl.ds(0, 16)] = lane0
        for g in range(1, D // 16):
            ones_v[r, pl.ds(g * 16, 16)] = zero16
        return carry
    lax.fori_loop(0, K, init_ones, None)

    pltpu.sync_copy(zrow_hbm, dacc.at[pl.ds(s * RPT, RPT)])
    plsc.subcore_barrier()

    def sup(cs, carry):
        pltpu.sync_copy(dst_hbm.at[wid, pl.ds(cs * SUP, SUP)], dst_v)

        def chunk(cj, inner):
            pltpu.sync_copy(ones_v, dacc.at[dst_v.at[cj]], add=True)
            return inner
        lax.fori_loop(0, SUP, chunk, None)
        return carry
    lax.fori_loop(0, NSUP, sup, None)

    plsc.subcore_barrier()
    pltpu.sync_copy(dacc.at[pl.ds(s * RPT, RPT)],
                    odeg_hbm.at[c, pl.ds(s * RPT, RPT)])


_BR = 1000


def _dinv_of(pdeg_ref):
    deg = pdeg_ref[0, :, 0] + pdeg_ref[1, :, 0] + 1.0
    return lax.rsqrt(deg)


def _tc_prep_body(x_ref, px2_ref, pdeg_ref, mask_ref, o_ref):
    dinv = _dinv_of(pdeg_ref)
    x2 = px2_ref[0] + px2_ref[1]
    m = mask_ref[0, 0]
    xt = jnp.where(m[:, None] > 0, x2, x_ref[...])
    o_ref[...] = dinv[:, None] * xt


def _tc_conv_body(pm_ref, aprev_ref, pdeg_ref, w_ref, b_ref, o_ref):
    dinv = _dinv_of(pdeg_ref)
    pre = dinv[:, None] * (pm_ref[0] + pm_ref[1] + aprev_ref[...])
    z = jnp.dot(pre, w_ref[...], preferred_element_type=_f32) + b_ref[...]
    o_ref[...] = dinv[:, None] * jnp.maximum(z, 0.0)


def _tc_final_body(pm_ref, aprev_ref, pdeg_ref, w3_ref, b3_ref, batch_ref,
                   wm1_ref, bm1_ref, wm2_ref, bm2_ref, o_ref, g_acc):
    i = pl.program_id(0)
    dinv = _dinv_of(pdeg_ref)
    pre = dinv[:, None] * (pm_ref[0] + pm_ref[1] + aprev_ref[...])
    z3 = jnp.dot(pre, w3_ref[...], preferred_element_type=_f32) + b3_ref[...]
    bt = batch_ref[0, 0]
    oh = (bt[:, None] ==
          lax.broadcasted_iota(jnp.int32, (1, 64), 1)).astype(_f32)
    contrib = lax.dot_general(oh, z3, (((0,), (0,)), ((), ())),
                              preferred_element_type=_f32)

    @pl.when(i == 0)
    def _():
        g_acc[...] = contrib

    @pl.when(i > 0)
    def _():
        g_acc[...] = g_acc[...] + contrib

    @pl.when(i == pl.num_programs(0) - 1)
    def _():
        h = jnp.maximum(
            jnp.dot(g_acc[...], wm1_ref[...], preferred_element_type=_f32)
            + bm1_ref[...], 0.0)
        o_ref[...] = (jnp.dot(h, wm2_ref[...], preferred_element_type=_f32)
                      + bm2_ref[...])


def _row_spec():
    return pl.BlockSpec((_BR, D), lambda i: (i, 0))


def _part_spec(width):
    return pl.BlockSpec((NC, _BR, width), lambda i: (0, i, 0))


def _i32row_spec():
    return pl.BlockSpec((1, 1, _BR), lambda i: (i, 0, 0))


def _full(shape):
    return pl.BlockSpec(shape, lambda i: tuple(0 for _ in shape))


def _tc_prep(x, px2, pdeg, mask3):
    return pl.pallas_call(
        _tc_prep_body,
        grid=(N // _BR,),
        in_specs=[_row_spec(), _part_spec(D), _part_spec(D), _i32row_spec()],
        out_specs=_row_spec(),
        out_shape=jax.ShapeDtypeStruct((N, D), _f32),
    )(x, px2, pdeg, mask3)


def _tc_conv(pm, aprev, pdeg, w, b2d):
    return pl.pallas_call(
        _tc_conv_body,
        grid=(N // _BR,),
        in_specs=[_part_spec(D), _row_spec(), _part_spec(D),
                  _full((D, D)), _full((1, D))],
        out_specs=_row_spec(),
        out_shape=jax.ShapeDtypeStruct((N, D), _f32),
    )(pm, aprev, pdeg, w, b2d)


def _tc_final(pm, aprev, pdeg, w3, b3, batch3, wm1, bm1, wm2, bm2):
    return pl.pallas_call(
        _tc_final_body,
        grid=(N // _BR,),
        in_specs=[_part_spec(D), _row_spec(), _part_spec(D),
                  _full((D, D)), _full((1, D)), _i32row_spec(),
                  _full((D, D)), _full((1, D)), _full((D, 10)),
                  _full((1, 10))],
        out_specs=_full((64, 10)),
        out_shape=jax.ShapeDtypeStruct((64, 10), _f32),
        scratch_shapes=[pltpu.VMEM((64, D), _f32)],
    )(pm, aprev, pdeg, w3, b3, batch3, wm1, bm1, wm2, bm2)


def kernel(x, edge_index, supernode_mask, edge_mask, batch,
           W1, b1, W2, b2, W3, b3, Wm1, bm1, Wm2, bm2):
    pad = EPAD - E
    src3 = jnp.concatenate(
        [edge_index[0], jnp.zeros((pad,), jnp.int32)]).reshape(NW, NCHUNK, K)
    dst3 = jnp.concatenate(
        [edge_index[1], jnp.full((pad,), N, jnp.int32)]).reshape(NW, NCHUNK, K)
    w3 = jnp.concatenate(
        [edge_mask, jnp.zeros((pad,), _f32)]).reshape(NW, NCHUNK, K)
    mask3 = supernode_mask.astype(jnp.int32).reshape(N // _BR, 1, _BR)
    batch3 = batch.reshape(N // _BR, 1, _BR)
    zrow = jnp.zeros((RPT, D), _f32)
    b1r, b2r, b3r = b1.reshape(1, D), b2.reshape(1, D), b3.reshape(1, D)
    bm1r, bm2r = bm1.reshape(1, D), bm2.reshape(1, 10)

    px2 = _sc_pass1(x, src3, dst3, w3, zrow)
    pdeg = _sc_deg(dst3, zrow)
    a1 = _tc_prep(x, px2, pdeg, mask3)
    m1 = _sc_agg_p(a1, src3, dst3, zrow)
    a2 = _tc_conv(m1, a1, pdeg, W1, b1r)
    m2 = _sc_agg_p(a2, src3, dst3, zrow)
    a3 = _tc_conv(m2, a2, pdeg, W2, b2r)
    m3 = _sc_agg_p(a3, src3, dst3, zrow)
    return _tc_final(m3, a3, pdeg, W3, b3r, batch3, Wm1, bm1r, Wm2, bm2r)
